# baseline (speedup 1.0000x reference)
import functools

import jax
import jax.numpy as jnp
from jax import lax
from jax.experimental import pallas as pl
from jax.experimental.pallas import tpu as pltpu
from jax.experimental.pallas import tpu_sc as plsc

VOCAB = 1000000
EMBED = 128
BATCH = 4096

_NC = 2
_NS = 16
_NW = _NC * _NS
_BPW = BATCH // _NW
_CH = _BPW // 2

_mesh = plsc.VectorSubcoreMesh(
    core_axis_name="c", subcore_axis_name="s", num_cores=_NC, num_subcores=_NS
)


@functools.partial(
    pl.kernel,
    out_type=(
        jax.ShapeDtypeStruct((BATCH, EMBED), jnp.float32),
        jax.ShapeDtypeStruct((BATCH, EMBED), jnp.float32),
    ),
    mesh=_mesh,
    scratch_types=[
        pltpu.VMEM((_BPW,), jnp.int32),
        pltpu.VMEM((_BPW,), jnp.int32),
        pltpu.VMEM((_BPW, EMBED), jnp.float32),
        pltpu.VMEM((_BPW, EMBED), jnp.float32),
        pltpu.SemaphoreType.DMA,
        pltpu.SemaphoreType.DMA,
        pltpu.SemaphoreType.DMA,
        pltpu.SemaphoreType.DMA,
        pltpu.SemaphoreType.DMA,
        pltpu.SemaphoreType.DMA,
    ],
)
def _sc_gather(target_hbm, context_hbm, in_tab_hbm, out_tab_hbm,
               in_emb_hbm, out_emb_hbm,
               tgt_idx_v, ctx_idx_v, in_rows_v, out_rows_v,
               sem_ia, sem_ib, sem_a0, sem_a1, sem_b0, sem_b1):
    wid = lax.axis_index("s") * _NC + lax.axis_index("c")
    base = wid * _BPW
    ia = pltpu.async_copy(target_hbm.at[pl.ds(base, _BPW)], tgt_idx_v, sem_ia)
    ib = pltpu.async_copy(context_hbm.at[pl.ds(base, _BPW)], ctx_idx_v, sem_ib)
    ia.wait()
    ga0 = pltpu.async_copy(in_tab_hbm.at[tgt_idx_v.at[pl.ds(0, _CH)]],
                           in_rows_v.at[pl.ds(0, _CH)], sem_a0)
    ga1 = pltpu.async_copy(in_tab_hbm.at[tgt_idx_v.at[pl.ds(_CH, _CH)]],
                           in_rows_v.at[pl.ds(_CH, _CH)], sem_a1)
    ib.wait()
    gb0 = pltpu.async_copy(out_tab_hbm.at[ctx_idx_v.at[pl.ds(0, _CH)]],
                           out_rows_v.at[pl.ds(0, _CH)], sem_b0)
    gb1 = pltpu.async_copy(out_tab_hbm.at[ctx_idx_v.at[pl.ds(_CH, _CH)]],
                           out_rows_v.at[pl.ds(_CH, _CH)], sem_b1)
    ga0.wait()
    wa0 = pltpu.async_copy(in_rows_v.at[pl.ds(0, _CH)],
                           in_emb_hbm.at[pl.ds(base, _CH)], sem_a0)
    ga1.wait()
    wa1 = pltpu.async_copy(in_rows_v.at[pl.ds(_CH, _CH)],
                           in_emb_hbm.at[pl.ds(base + _CH, _CH)], sem_a1)
    gb0.wait()
    wb0 = pltpu.async_copy(out_rows_v.at[pl.ds(0, _CH)],
                           out_emb_hbm.at[pl.ds(base, _CH)], sem_b0)
    gb1.wait()
    wb1 = pltpu.async_copy(out_rows_v.at[pl.ds(_CH, _CH)],
                           out_emb_hbm.at[pl.ds(base + _CH, _CH)], sem_b1)
    wa0.wait()
    wa1.wait()
    wb0.wait()
    wb1.wait()


_BM = 512
_NBLK = BATCH // _BM
_NBUF = 3


def _mm_body(a_ref, b_ref, o_hbm, obuf, sem0, sem1, sem2):
    sems = [sem0, sem1, sem2]
    b_bf = b_ref[...].astype(jnp.bfloat16)
    for i in range(_NBLK):
        k = i % _NBUF
        dst = o_hbm.at[pl.ds(i * _BM, _BM), :]
        cp = pltpu.make_async_copy(obuf.at[k], dst, sems[k])
        if i >= _NBUF:
            prev = o_hbm.at[pl.ds((i - _NBUF) * _BM, _BM), :]
            pltpu.make_async_copy(obuf.at[k], prev, sems[k]).wait()
        obuf[k] = lax.dot_general(
            a_ref[pl.ds(i * _BM, _BM), :].astype(jnp.bfloat16), b_bf,
            dimension_numbers=(((1,), (1,)), ((), ())),
            preferred_element_type=jnp.float32,
        )
        cp.start()
    for i in range(_NBLK - _NBUF, _NBLK):
        k = i % _NBUF
        dst = o_hbm.at[pl.ds(i * _BM, _BM), :]
        pltpu.make_async_copy(obuf.at[k], dst, sems[k]).wait()


_matmul = pl.pallas_call(
    _mm_body,
    in_specs=[
        pl.BlockSpec((BATCH, EMBED), lambda: (0, 0)),
        pl.BlockSpec((BATCH, EMBED), lambda: (0, 0)),
    ],
    out_specs=pl.BlockSpec(memory_space=pl.ANY),
    out_shape=jax.ShapeDtypeStruct((BATCH, BATCH), jnp.float32),
    scratch_shapes=[
        pltpu.VMEM((_NBUF, _BM, BATCH), jnp.float32),
        pltpu.SemaphoreType.DMA,
        pltpu.SemaphoreType.DMA,
        pltpu.SemaphoreType.DMA,
    ],
)


def kernel(target, context, in_table, out_table):
    target = target.astype(jnp.int32)
    context = context.astype(jnp.int32)
    in_embeds, out_embeds = _sc_gather(target, context, in_table, out_table)
    return _matmul(in_embeds, out_embeds)


# speedup vs baseline: 1.0135x; 1.0135x over previous
import functools

import jax
import jax.numpy as jnp
from jax import lax
from jax.experimental import pallas as pl
from jax.experimental.pallas import tpu as pltpu
from jax.experimental.pallas import tpu_sc as plsc

VOCAB = 1000000
EMBED = 128
BATCH = 4096

_NC = 2
_NS = 16
_NW = _NC * _NS
_BPW = BATCH // _NW
_NCH = 4
_CH = _BPW // _NCH

_mesh = plsc.VectorSubcoreMesh(
    core_axis_name="c", subcore_axis_name="s", num_cores=_NC, num_subcores=_NS
)


@functools.partial(
    pl.kernel,
    out_type=(
        jax.ShapeDtypeStruct((BATCH, EMBED), jnp.float32),
        jax.ShapeDtypeStruct((BATCH, EMBED), jnp.float32),
    ),
    mesh=_mesh,
    scratch_types=[
        pltpu.VMEM((_BPW,), jnp.int32),
        pltpu.VMEM((_BPW,), jnp.int32),
        pltpu.VMEM((_BPW, EMBED), jnp.float32),
        pltpu.VMEM((_BPW, EMBED), jnp.float32),
        pltpu.SemaphoreType.DMA,
        pltpu.SemaphoreType.DMA,
    ] + [pltpu.SemaphoreType.DMA] * (2 * _NCH),
)
def _sc_gather(target_hbm, context_hbm, in_tab_hbm, out_tab_hbm,
               in_emb_hbm, out_emb_hbm,
               tgt_idx_v, ctx_idx_v, in_rows_v, out_rows_v,
               sem_ia, sem_ib, *sems):
    wid = lax.axis_index("s") * _NC + lax.axis_index("c")
    base = wid * _BPW
    ia = pltpu.async_copy(target_hbm.at[pl.ds(base, _BPW)], tgt_idx_v, sem_ia)
    ib = pltpu.async_copy(context_hbm.at[pl.ds(base, _BPW)], ctx_idx_v, sem_ib)
    work = []
    ia.wait()
    for c in range(_NCH):
        g = pltpu.async_copy(
            in_tab_hbm.at[tgt_idx_v.at[pl.ds(c * _CH, _CH)]],
            in_rows_v.at[pl.ds(c * _CH, _CH)], sems[c])
        work.append((g, in_rows_v.at[pl.ds(c * _CH, _CH)],
                     in_emb_hbm.at[pl.ds(base + c * _CH, _CH)], sems[c]))
    ib.wait()
    for c in range(_NCH):
        g = pltpu.async_copy(
            out_tab_hbm.at[ctx_idx_v.at[pl.ds(c * _CH, _CH)]],
            out_rows_v.at[pl.ds(c * _CH, _CH)], sems[_NCH + c])
        work.append((g, out_rows_v.at[pl.ds(c * _CH, _CH)],
                     out_emb_hbm.at[pl.ds(base + c * _CH, _CH)], sems[_NCH + c]))
    writes = []
    for g, rows, dst, sem in work:
        g.wait()
        writes.append(pltpu.async_copy(rows, dst, sem))
    for w in writes:
        w.wait()


_BM = 512


def _mm_body(a_ref, b_ref, o_ref):
    o_ref[...] = lax.dot_general(
        a_ref[...].astype(jnp.bfloat16), b_ref[...].astype(jnp.bfloat16),
        dimension_numbers=(((1,), (1,)), ((), ())),
        preferred_element_type=jnp.float32,
    )


_matmul = pl.pallas_call(
    _mm_body,
    grid=(BATCH // _BM,),
    in_specs=[
        pl.BlockSpec((_BM, EMBED), lambda i: (i, 0)),
        pl.BlockSpec((BATCH, EMBED), lambda i: (0, 0)),
    ],
    out_specs=pl.BlockSpec((_BM, BATCH), lambda i: (i, 0)),
    out_shape=jax.ShapeDtypeStruct((BATCH, BATCH), jnp.float32),
)


def kernel(target, context, in_table, out_table):
    target = target.astype(jnp.int32)
    context = context.astype(jnp.int32)
    in_embeds, out_embeds = _sc_gather(target, context, in_table, out_table)
    return _matmul(in_embeds, out_embeds)


# speedup vs baseline: 1.0208x; 1.0071x over previous
import functools

import jax
import jax.numpy as jnp
from jax import lax
from jax.experimental import pallas as pl
from jax.experimental.pallas import tpu as pltpu
from jax.experimental.pallas import tpu_sc as plsc

VOCAB = 1000000
EMBED = 128
BATCH = 4096

_NC = 2
_NS = 16
_NW = _NC * _NS
_BPW = BATCH // _NW
_NCH = 2
_CH = _BPW // _NCH

_mesh = plsc.VectorSubcoreMesh(
    core_axis_name="c", subcore_axis_name="s", num_cores=_NC, num_subcores=_NS
)


@functools.partial(
    pl.kernel,
    out_type=(
        jax.ShapeDtypeStruct((BATCH, EMBED), jnp.float32),
        jax.ShapeDtypeStruct((BATCH, EMBED), jnp.float32),
    ),
    mesh=_mesh,
    scratch_types=[
        pltpu.VMEM((_BPW,), jnp.int32),
        pltpu.VMEM((_BPW,), jnp.int32),
        pltpu.VMEM((_BPW, EMBED), jnp.float32),
        pltpu.VMEM((_BPW, EMBED), jnp.float32),
        pltpu.SemaphoreType.DMA,
        pltpu.SemaphoreType.DMA,
    ] + [pltpu.SemaphoreType.DMA] * (2 * _NCH),
)
def _sc_gather(target_hbm, context_hbm, in_tab_hbm, out_tab_hbm,
               in_emb_hbm, out_emb_hbm,
               tgt_idx_v, ctx_idx_v, in_rows_v, out_rows_v,
               sem_ia, sem_ib, *sems):
    wid = lax.axis_index("s") * _NC + lax.axis_index("c")
    base = wid * _BPW
    ia = pltpu.async_copy(target_hbm.at[pl.ds(base, _BPW)], tgt_idx_v, sem_ia)
    ib = pltpu.async_copy(context_hbm.at[pl.ds(base, _BPW)], ctx_idx_v, sem_ib)
    work = []
    ia.wait()
    for c in range(_NCH):
        g = pltpu.async_copy(
            in_tab_hbm.at[tgt_idx_v.at[pl.ds(c * _CH, _CH)]],
            in_rows_v.at[pl.ds(c * _CH, _CH)], sems[c])
        work.append((g, in_rows_v.at[pl.ds(c * _CH, _CH)],
                     in_emb_hbm.at[pl.ds(base + c * _CH, _CH)], sems[c]))
    ib.wait()
    for c in range(_NCH):
        g = pltpu.async_copy(
            out_tab_hbm.at[ctx_idx_v.at[pl.ds(c * _CH, _CH)]],
            out_rows_v.at[pl.ds(c * _CH, _CH)], sems[_NCH + c])
        work.append((g, out_rows_v.at[pl.ds(c * _CH, _CH)],
                     out_emb_hbm.at[pl.ds(base + c * _CH, _CH)], sems[_NCH + c]))
    writes = []
    for g, rows, dst, sem in work:
        g.wait()
        writes.append(pltpu.async_copy(rows, dst, sem))
    for w in writes:
        w.wait()


_BM = 512


def _mm_body(a_ref, b_ref, o_ref):
    o_ref[...] = lax.dot_general(
        a_ref[...].astype(jnp.bfloat16), b_ref[...].astype(jnp.bfloat16),
        dimension_numbers=(((1,), (1,)), ((), ())),
        preferred_element_type=jnp.float32,
    )


_matmul = pl.pallas_call(
    _mm_body,
    grid=(BATCH // _BM,),
    in_specs=[
        pl.BlockSpec((_BM, EMBED), lambda i: (i, 0)),
        pl.BlockSpec((BATCH, EMBED), lambda i: (0, 0)),
    ],
    out_specs=pl.BlockSpec((_BM, BATCH), lambda i: (i, 0)),
    out_shape=jax.ShapeDtypeStruct((BATCH, BATCH), jnp.float32),
)


def kernel(target, context, in_table, out_table):
    target = target.astype(jnp.int32)
    context = context.astype(jnp.int32)
    in_embeds, out_embeds = _sc_gather(target, context, in_table, out_table)
    return _matmul(in_embeds, out_embeds)
